# Initial kernel scaffold; baseline (speedup 1.0000x reference)
import jax
import jax.numpy as jnp
from jax.experimental import pallas as pl


def kernel(x, Wr, W1, b1, W2, b2, fb_w1, fb_b1, fb_w2, fb_b2):
    raise NotImplementedError("write your pallas kernel here")


# speedup vs baseline: 2.4689x; 2.4689x over previous
import jax
import jax.numpy as jnp
from jax.experimental import pallas as pl
from jax.experimental.pallas import tpu as pltpu

def _gelu_exact(v):
    s = v * 0.7071067811865476
    a = jnp.abs(s)
    t = 1.0 / (1.0 + 0.3275911 * a)
    poly = t * (0.254829592 + t * (-0.284496736 + t * (
        1.421413741 + t * (-1.453152027 + t * 1.061405429))))
    erf_abs = 1.0 - poly * jnp.exp(-a * a)
    erf = jnp.sign(s) * erf_abs
    return 0.5 * v * (1.0 + erf)


_TOP_K = 2
_CAP_FACTOR, _MIN_CAP = 1.25, 4
_Z_COEF, _LB_COEF = 0.001, 0.01


def _router_fb_kernel(x_ref, wr_ref, fw1_ref, fb1_ref, fw2_ref, fb2_ref,
                      yfb_ref, e1_ref, e2_ref, w1_ref, w2_ref,
                      imp_ref, load_ref, z_ref):
    i = pl.program_id(0)
    x = x_ref[...]
    logits = jnp.dot(x, wr_ref[...], preferred_element_type=jnp.float32)
    m = jnp.max(logits, axis=-1, keepdims=True)
    ex = jnp.exp(logits - m)
    se = jnp.sum(ex, axis=-1, keepdims=True)
    probs = ex / se
    lse = jnp.log(se[:, 0]) + m[:, 0]

    n_e = probs.shape[-1]
    cols = jax.lax.broadcasted_iota(jnp.int32, probs.shape, 1)
    p1 = jnp.max(probs, axis=-1)
    i1 = jnp.min(jnp.where(probs == p1[:, None], cols, n_e), axis=-1)
    masked = jnp.where(cols == i1[:, None], -1.0, probs)
    p2 = jnp.max(masked, axis=-1)
    i2 = jnp.min(jnp.where(masked == p2[:, None], cols, n_e), axis=-1)
    denom = jnp.clip(p1 + p2, 1e-9, None)

    e1_ref[...] = i1.astype(jnp.int32)
    e2_ref[...] = i2.astype(jnp.int32)
    w1_ref[...] = p1 / denom
    w2_ref[...] = p2 / denom

    h = jnp.dot(x, fw1_ref[...], preferred_element_type=jnp.float32) + fb1_ref[...]
    h = _gelu_exact(h)
    yfb_ref[...] = jnp.dot(h, fw2_ref[...], preferred_element_type=jnp.float32) + fb2_ref[...]

    imp = jnp.sum(probs, axis=0)[None, :]
    ld = jnp.sum((cols == i1[:, None]).astype(jnp.float32), axis=0)[None, :]
    zz = jnp.sum(lse * lse).reshape(1, 1)

    @pl.when(i == 0)
    def _():
        imp_ref[...] = jnp.zeros_like(imp_ref)
        load_ref[...] = jnp.zeros_like(load_ref)
        z_ref[...] = jnp.zeros_like(z_ref)

    imp_ref[...] += imp
    load_ref[...] += ld
    z_ref[...] += zz


def _expert_ffn_kernel(ws_ref, xd_ref, w1_ref, b1_ref, w2_ref, b2_ref, out_ref):
    x = xd_ref[...]
    h = jnp.dot(x, w1_ref[0], preferred_element_type=jnp.float32) + b1_ref[0]
    h = _gelu_exact(h)
    y = jnp.dot(h, w2_ref[0], preferred_element_type=jnp.float32) + b2_ref[0]
    ws = ws_ref[...]
    out_ref[...] = y * ws[0, 0, :, None]


def kernel(x, Wr, W1, b1, W2, b2, fb_w1, fb_b1, fb_w2, fb_b2):
    Bc, Sc, D = x.shape
    T = Bc * Sc
    E = Wr.shape[1]
    DFF = W1.shape[2]
    cap = max(int(_CAP_FACTOR * (T * _TOP_K / E)), _MIN_CAP)
    x_flat = x.reshape(T, D)

    BT = 512
    nb = T // BT
    router = pl.pallas_call(
        _router_fb_kernel,
        grid=(nb,),
        in_specs=[
            pl.BlockSpec((BT, D), lambda i: (i, 0)),
            pl.BlockSpec((D, E), lambda i: (0, 0)),
            pl.BlockSpec((D, DFF), lambda i: (0, 0)),
            pl.BlockSpec((1, DFF), lambda i: (0, 0)),
            pl.BlockSpec((DFF, D), lambda i: (0, 0)),
            pl.BlockSpec((1, D), lambda i: (0, 0)),
        ],
        out_specs=[
            pl.BlockSpec((BT, D), lambda i: (i, 0)),
            pl.BlockSpec((BT,), lambda i: (i,)),
            pl.BlockSpec((BT,), lambda i: (i,)),
            pl.BlockSpec((BT,), lambda i: (i,)),
            pl.BlockSpec((BT,), lambda i: (i,)),
            pl.BlockSpec((1, E), lambda i: (0, 0)),
            pl.BlockSpec((1, E), lambda i: (0, 0)),
            pl.BlockSpec((1, 1), lambda i: (0, 0)),
        ],
        out_shape=[
            jax.ShapeDtypeStruct((T, D), jnp.float32),
            jax.ShapeDtypeStruct((T,), jnp.int32),
            jax.ShapeDtypeStruct((T,), jnp.int32),
            jax.ShapeDtypeStruct((T,), jnp.float32),
            jax.ShapeDtypeStruct((T,), jnp.float32),
            jax.ShapeDtypeStruct((1, E), jnp.float32),
            jax.ShapeDtypeStruct((1, E), jnp.float32),
            jax.ShapeDtypeStruct((1, 1), jnp.float32),
        ],
        compiler_params=pltpu.CompilerParams(
            dimension_semantics=("arbitrary",)),
    )
    y_fb, e1, e2, w1n, w2n, imp, load, zsum = router(
        x_flat, Wr, fb_w1, fb_b1.reshape(1, DFF), fb_w2, fb_b2.reshape(1, D))

    tok_ids = jnp.arange(T, dtype=jnp.int32)

    def rbody(fmask, e):
        m0 = e1 == e
        m1 = e2 == e
        kept = (m0 | m1) & jnp.logical_not(fmask)
        ki = jnp.cumsum(kept.astype(jnp.int32)) - kept.astype(jnp.int32)
        process = kept & (ki < cap)
        overflow = kept & (ki >= cap)
        scatter_idx = jnp.where(process, ki, cap)
        buf = jnp.zeros((cap + 1,), jnp.int32).at[scatter_idx].set(tok_ids)
        n = jnp.sum(process.astype(jnp.int32))
        return fmask | overflow, (buf[:cap], n)

    fmask, (TI, N) = jax.lax.scan(
        rbody, jnp.zeros((T,), bool), jnp.arange(E, dtype=jnp.int32))

    eids = jnp.arange(E, dtype=jnp.int32)[:, None]
    w_slot = jnp.where(e1[TI] == eids, w1n[TI], w2n[TI])
    w_slot = jnp.where(jnp.arange(cap)[None, :] < N[:, None], w_slot, 0.0)
    x_d = x_flat[TI.reshape(-1)]

    BTB = 256
    ncb = cap // BTB
    expert_ffn = pl.pallas_call(
        _expert_ffn_kernel,
        grid=(E, ncb),
        in_specs=[
            pl.BlockSpec((1, 1, BTB), lambda e, c: (e, 0, c)),
            pl.BlockSpec((BTB, D), lambda e, c: (e * ncb + c, 0)),
            pl.BlockSpec((1, D, DFF), lambda e, c: (e, 0, 0)),
            pl.BlockSpec((1, 1, DFF), lambda e, c: (e, 0, 0)),
            pl.BlockSpec((1, DFF, D), lambda e, c: (e, 0, 0)),
            pl.BlockSpec((1, 1, D), lambda e, c: (e, 0, 0)),
        ],
        out_specs=pl.BlockSpec((BTB, D), lambda e, c: (e * ncb + c, 0)),
        out_shape=jax.ShapeDtypeStruct((E * cap, D), jnp.float32),
        compiler_params=pltpu.CompilerParams(
            dimension_semantics=("arbitrary", "arbitrary")),
    )
    y_d = expert_ffn(w_slot.reshape(E, 1, cap), x_d,
                     W1, b1.reshape(E, 1, DFF), W2, b2.reshape(E, 1, D))

    y_flat = jnp.zeros((T, D), jnp.float32).at[TI.reshape(-1)].add(y_d)
    y_flat = jnp.where(fmask[:, None], y_fb, y_flat)
    y = y_flat.reshape(Bc, Sc, D)

    z_loss = (zsum[0, 0] / T) * _Z_COEF
    impv = imp[0]
    loadv = load[0]
    impv = impv / jnp.clip(impv.sum(), 1e-9, None)
    loadv = loadv / jnp.clip(loadv.sum(), 1e-9, None)
    lb_loss = jnp.sum(impv * loadv) * (E ** 2) * _LB_COEF
    return (y, z_loss, lb_loss)


# speedup vs baseline: 8.1683x; 3.3085x over previous
import jax
import jax.numpy as jnp
from jax.experimental import pallas as pl
from jax.experimental.pallas import tpu as pltpu


def _gelu_exact(v):
    s = v * 0.7071067811865476
    a = jnp.abs(s)
    t = 1.0 / (1.0 + 0.3275911 * a)
    poly = t * (0.254829592 + t * (-0.284496736 + t * (
        1.421413741 + t * (-1.453152027 + t * 1.061405429))))
    erf_abs = 1.0 - poly * jnp.exp(-a * a)
    erf = jnp.sign(s) * erf_abs
    return 0.5 * v * (1.0 + erf)


_TOP_K = 2
_CAP_FACTOR, _MIN_CAP = 1.25, 4
_Z_COEF, _LB_COEF = 0.001, 0.01


def _router_kernel(x_ref, wr_ref,
                   e1_ref, e2_ref, w1_ref, w2_ref, r1_ref, r2_ref,
                   loads_ref, imp_ref, load_ref, z_ref, cnt_ref):
    i = pl.program_id(0)
    x = x_ref[...]
    logits = jnp.dot(x, wr_ref[...], preferred_element_type=jnp.float32)
    m = jnp.max(logits, axis=-1, keepdims=True)
    ex = jnp.exp(logits - m)
    se = jnp.sum(ex, axis=-1, keepdims=True)
    probs = ex / se
    lse = jnp.log(se[:, 0]) + m[:, 0]

    n_e = probs.shape[-1]
    bt = probs.shape[0]
    cols = jax.lax.broadcasted_iota(jnp.int32, probs.shape, 1)
    p1 = jnp.max(probs, axis=-1)
    i1 = jnp.min(jnp.where(probs == p1[:, None], cols, n_e), axis=-1)
    masked = jnp.where(cols == i1[:, None], -1.0, probs)
    p2 = jnp.max(masked, axis=-1)
    i2 = jnp.min(jnp.where(masked == p2[:, None], cols, n_e), axis=-1)
    denom = jnp.clip(p1 + p2, 1e-9, None)

    e1_ref[...] = i1.astype(jnp.int32)
    e2_ref[...] = i2.astype(jnp.int32)
    w1_ref[...] = p1 / denom
    w2_ref[...] = p2 / denom

    onehot1 = (cols == i1[:, None]).astype(jnp.float32)
    onehot2 = (cols == i2[:, None]).astype(jnp.float32)
    onehot = onehot1 + onehot2
    rows_i = jax.lax.broadcasted_iota(jnp.int32, (bt, bt), 0)
    cols_i = jax.lax.broadcasted_iota(jnp.int32, (bt, bt), 1)
    tri = (rows_i > cols_i).astype(jnp.float32)
    prior = jnp.dot(tri, onehot, preferred_element_type=jnp.float32)

    @pl.when(i == 0)
    def _():
        cnt_ref[...] = jnp.zeros_like(cnt_ref)
        imp_ref[...] = jnp.zeros_like(imp_ref)
        load_ref[...] = jnp.zeros_like(load_ref)
        z_ref[...] = jnp.zeros_like(z_ref)

    base = cnt_ref[...]
    rank_e = prior + base
    r1_ref[...] = jnp.sum(rank_e * onehot1, axis=1).astype(jnp.int32)
    r2_ref[...] = jnp.sum(rank_e * onehot2, axis=1).astype(jnp.int32)
    cnt_new = base + jnp.sum(onehot, axis=0, keepdims=True)
    cnt_ref[...] = cnt_new
    loads_ref[...] = cnt_new

    imp_ref[...] += jnp.sum(probs, axis=0)[None, :]
    load_ref[...] += jnp.sum(onehot1, axis=0)[None, :]
    z_ref[...] += jnp.sum(lse * lse).reshape(1, 1)


def _expert_ffn_kernel(ws_ref, xd_ref, w1_ref, b1_ref, w2_ref, b2_ref, out_ref):
    x = xd_ref[...]
    h = jnp.dot(x, w1_ref[0], preferred_element_type=jnp.float32) + b1_ref[0]
    h = _gelu_exact(h)
    y = jnp.dot(h, w2_ref[0], preferred_element_type=jnp.float32) + b2_ref[0]
    ws = ws_ref[...]
    out_ref[...] = y * ws[0, 0, :, None]


def _fb_ffn_kernel(x_ref, fw1_ref, fb1_ref, fw2_ref, fb2_ref, out_ref):
    x = x_ref[...]
    h = jnp.dot(x, fw1_ref[...], preferred_element_type=jnp.float32) + fb1_ref[...]
    h = _gelu_exact(h)
    out_ref[...] = jnp.dot(h, fw2_ref[...], preferred_element_type=jnp.float32) + fb2_ref[...]


def kernel(x, Wr, W1, b1, W2, b2, fb_w1, fb_b1, fb_w2, fb_b2):
    Bc, Sc, D = x.shape
    T = Bc * Sc
    E = Wr.shape[1]
    DFF = W1.shape[2]
    cap = max(int(_CAP_FACTOR * (T * _TOP_K / E)), _MIN_CAP)
    x_flat = x.reshape(T, D)

    BT = 512
    nb = T // BT
    router = pl.pallas_call(
        _router_kernel,
        grid=(nb,),
        in_specs=[
            pl.BlockSpec((BT, D), lambda i: (i, 0)),
            pl.BlockSpec((D, E), lambda i: (0, 0)),
        ],
        out_specs=[
            pl.BlockSpec((BT,), lambda i: (i,)),
            pl.BlockSpec((BT,), lambda i: (i,)),
            pl.BlockSpec((BT,), lambda i: (i,)),
            pl.BlockSpec((BT,), lambda i: (i,)),
            pl.BlockSpec((BT,), lambda i: (i,)),
            pl.BlockSpec((BT,), lambda i: (i,)),
            pl.BlockSpec((1, E), lambda i: (0, 0)),
            pl.BlockSpec((1, E), lambda i: (0, 0)),
            pl.BlockSpec((1, E), lambda i: (0, 0)),
            pl.BlockSpec((1, 1), lambda i: (0, 0)),
        ],
        out_shape=[
            jax.ShapeDtypeStruct((T,), jnp.int32),
            jax.ShapeDtypeStruct((T,), jnp.int32),
            jax.ShapeDtypeStruct((T,), jnp.float32),
            jax.ShapeDtypeStruct((T,), jnp.float32),
            jax.ShapeDtypeStruct((T,), jnp.int32),
            jax.ShapeDtypeStruct((T,), jnp.int32),
            jax.ShapeDtypeStruct((1, E), jnp.float32),
            jax.ShapeDtypeStruct((1, E), jnp.float32),
            jax.ShapeDtypeStruct((1, E), jnp.float32),
            jax.ShapeDtypeStruct((1, 1), jnp.float32),
        ],
        scratch_shapes=[pltpu.VMEM((1, E), jnp.float32)],
        compiler_params=pltpu.CompilerParams(
            dimension_semantics=("arbitrary",)),
    )
    e1, e2, w1n, w2n, rank1, rank2, loads_f, imp, load, zsum = router(x_flat, Wr)

    loads = loads_f[0].astype(jnp.int32)
    no_ovf = jnp.max(loads) <= cap
    tok = jnp.arange(T, dtype=jnp.int32)
    SEN = E * cap

    def _fast():
        fmask = jnp.zeros((T,), bool)
        return fmask, e1 * cap + rank1, e2 * cap + rank2, loads

    def _slow():
        def rbody(carry, e):
            fmask, pos1, pos2 = carry
            m0 = e1 == e
            m1 = e2 == e
            kept = (m0 | m1) & jnp.logical_not(fmask)
            k32 = kept.astype(jnp.int32)
            ki = jnp.cumsum(k32) - k32
            process = kept & (ki < cap)
            overflow = kept & (ki >= cap)
            p = e * cap + ki
            pos1 = jnp.where(process & m0, p, pos1)
            pos2 = jnp.where(process & m1, p, pos2)
            n = jnp.sum(process.astype(jnp.int32))
            return (fmask | overflow, pos1, pos2), n

        init = (jnp.zeros((T,), bool),
                jnp.full((T,), SEN, jnp.int32),
                jnp.full((T,), SEN, jnp.int32))
        (fmask, pos1, pos2), N = jax.lax.scan(
            rbody, init, jnp.arange(E, dtype=jnp.int32))
        return fmask, pos1, pos2, N

    fmask, pos1, pos2, N = jax.lax.cond(no_ovf, _fast, _slow)

    buf = jnp.zeros((SEN + 1,), jnp.int32).at[pos1].set(tok).at[pos2].set(tok)
    TI = buf[:SEN].reshape(E, cap)
    eids = jnp.arange(E, dtype=jnp.int32)[:, None]
    w_slot = jnp.where(e1[TI] == eids, w1n[TI], w2n[TI])
    w_slot = jnp.where(jnp.arange(cap)[None, :] < N[:, None], w_slot, 0.0)
    x_d = x_flat[TI.reshape(-1)]

    BTB = 256
    ncb = cap // BTB
    expert_ffn = pl.pallas_call(
        _expert_ffn_kernel,
        grid=(E, ncb),
        in_specs=[
            pl.BlockSpec((1, 1, BTB), lambda e, c: (e, 0, c)),
            pl.BlockSpec((BTB, D), lambda e, c: (e * ncb + c, 0)),
            pl.BlockSpec((1, D, DFF), lambda e, c: (e, 0, 0)),
            pl.BlockSpec((1, 1, DFF), lambda e, c: (e, 0, 0)),
            pl.BlockSpec((1, DFF, D), lambda e, c: (e, 0, 0)),
            pl.BlockSpec((1, 1, D), lambda e, c: (e, 0, 0)),
        ],
        out_specs=pl.BlockSpec((BTB, D), lambda e, c: (e * ncb + c, 0)),
        out_shape=jax.ShapeDtypeStruct((SEN, D), jnp.float32),
        compiler_params=pltpu.CompilerParams(
            dimension_semantics=("arbitrary", "arbitrary")),
    )
    y_d = expert_ffn(w_slot.reshape(E, 1, cap), x_d,
                     W1, b1.reshape(E, 1, DFF), W2, b2.reshape(E, 1, D))

    y_d_pad = jnp.concatenate([y_d, jnp.zeros((1, D), jnp.float32)], axis=0)
    y_comb = y_d_pad[pos1] + y_d_pad[pos2]

    def _no_fb(yc):
        return yc

    def _with_fb(yc):
        fb = pl.pallas_call(
            _fb_ffn_kernel,
            grid=(nb,),
            in_specs=[
                pl.BlockSpec((BT, D), lambda i: (i, 0)),
                pl.BlockSpec((D, DFF), lambda i: (0, 0)),
                pl.BlockSpec((1, DFF), lambda i: (0, 0)),
                pl.BlockSpec((DFF, D), lambda i: (0, 0)),
                pl.BlockSpec((1, D), lambda i: (0, 0)),
            ],
            out_specs=pl.BlockSpec((BT, D), lambda i: (i, 0)),
            out_shape=jax.ShapeDtypeStruct((T, D), jnp.float32),
            compiler_params=pltpu.CompilerParams(
                dimension_semantics=("arbitrary",)),
        )
        y_fb = fb(x_flat, fb_w1, fb_b1.reshape(1, DFF), fb_w2, fb_b2.reshape(1, D))
        return jnp.where(fmask[:, None], y_fb, yc)

    y_flat = jax.lax.cond(no_ovf, _no_fb, _with_fb, y_comb)
    y = y_flat.reshape(Bc, Sc, D)

    z_loss = (zsum[0, 0] / T) * _Z_COEF
    impv = imp[0]
    loadv = load[0]
    impv = impv / jnp.clip(impv.sum(), 1e-9, None)
    loadv = loadv / jnp.clip(loadv.sum(), 1e-9, None)
    lb_loss = jnp.sum(impv * loadv) * (E ** 2) * _LB_COEF
    return (y, z_loss, lb_loss)
